# baseline (speedup 1.0000x reference)
import functools

import jax
import jax.numpy as jnp
from jax import lax
from jax.experimental import pallas as pl
from jax.experimental.pallas import tpu as pltpu
from jax.experimental.pallas import tpu_sc as plsc

VOCAB = 1000000
EMBED_DIM = 64
BATCH = 4096
HIST = 200

NW = 32
BT = BATCH // 128
NB = 2


def _body(idx_hbm, tab_hbm, out_hbm, idxv, idxh, rows, tbufs, gsems, csems):
    wid = lax.axis_index("s") * 2 + lax.axis_index("c")

    pltpu.sync_copy(idx_hbm.at[wid], idxv)

    @pl.loop(0, HIST)
    def halve(h):
        for k in range(8):
            v = idxv[h, pl.ds(k * 16, 16)]
            idxh[h, pl.ds(k * 16, 16)] = jnp.right_shift(v, 1)

    iota16 = lax.iota(jnp.int32, 16)

    def gather_start(h, slot):
        pltpu.async_copy(tab_hbm.at[idxh.at[h]], rows[slot], gsems[slot])

    def gather_wait(slot):
        pltpu.make_async_copy(tab_hbm.at[idxh.at[0]], rows[slot],
                              gsems[slot]).wait()

    def copyout_start(h, slot):
        for ct in range(8):
            pltpu.async_copy(tbufs[slot].at[ct], out_hbm.at[h, ct, wid],
                             csems[slot])

    def copyout_wait(slot):
        for ct in range(8):
            pltpu.make_async_copy(tbufs[slot].at[ct], out_hbm.at[0, ct, 0],
                                  csems[slot]).wait()

    def transpose_block(h, slot):
        rv = rows[slot]
        tb = tbufs[slot]
        for g in range(8):
            iv = idxv[h, pl.ds(g * 16, 16)]
            brow = iota16 + (g * 16)
            bcol = jnp.bitwise_and(iv, 1) * 64
            for c in range(EMBED_DIM):
                v = plsc.load_gather(rv, [brow, bcol + c])
                tb[c // 8, c % 8, pl.ds(g * 16, 16)] = v

    for b in range(NB):
        gather_start(b, b)

    @pl.loop(0, HIST // NB)
    def group(g):
        base = g * NB
        for b in range(NB):
            h = base + b
            gather_wait(b)

            @pl.when(g > 0)
            def _():
                copyout_wait(b)

            transpose_block(h, b)
            copyout_start(h, b)

            @pl.when(g < HIST // NB - 1)
            def _():
                gather_start(h + NB, b)

    for b in range(NB):
        copyout_wait(b)


@functools.partial(
    pl.kernel,
    out_type=jax.ShapeDtypeStruct((HIST, 8, BT, 8, 128), jnp.float32),
    mesh=plsc.VectorSubcoreMesh(core_axis_name="c", subcore_axis_name="s"),
    compiler_params=pltpu.CompilerParams(needs_layout_passes=False),
    scratch_types=(
        [pltpu.VMEM((HIST, 128), jnp.int32),
         pltpu.VMEM((HIST, 128), jnp.int32)]
        + [pltpu.VMEM((128, 128), jnp.float32) for _ in range(NB)]
        + [pltpu.VMEM((8, 8, 128), jnp.float32) for _ in range(NB)]
        + [pltpu.SemaphoreType.DMA for _ in range(2 * NB)]
    ),
)
def _gather_kernel(idx_hbm, tab_hbm, out_hbm, idxv, idxh, *bufs):
    rows = bufs[:NB]
    tbufs = bufs[NB:2 * NB]
    gsems = bufs[2 * NB:3 * NB]
    csems = bufs[3 * NB:]
    _body(idx_hbm, tab_hbm, out_hbm, idxv, idxh, rows, tbufs, gsems, csems)


@jax.jit
def kernel(indices, table):
    idx3 = (indices.astype(jnp.int32).T
            .reshape(HIST, BT, 128).transpose(1, 0, 2))
    tab2 = table.reshape(VOCAB // 2, 2 * EMBED_DIM)
    o5 = _gather_kernel(idx3, tab2)
    return o5.transpose(2, 4, 0, 1, 3).reshape(BATCH, HIST, EMBED_DIM)


# speedup vs baseline: 1.8720x; 1.8720x over previous
import functools

import jax
import jax.numpy as jnp
from jax import lax
from jax.experimental import pallas as pl
from jax.experimental.pallas import tpu as pltpu
from jax.experimental.pallas import tpu_sc as plsc

VOCAB = 1000000
EMBED_DIM = 64
BATCH = 4096
HIST = 200

NW = 32
BT = BATCH // 128
NB = 2
RT_FULL = VOCAB // 128
RT_ITER = 246
TAIL_BASE = RT_FULL * 128


def _diag_vectors():
    iota16 = lax.iota(jnp.int32, 16)
    return iota16, [jnp.bitwise_and(iota16 + d, 15) for d in range(16)]


def _a_body(tab_hbm, pairs_hbm, slabs, pbufs, gsems, csems):
    wid = lax.axis_index("s") * 2 + lax.axis_index("c")
    iota16, P = _diag_vectors()
    halfsel = jnp.bitwise_and(iota16, 1) * 64

    def slab_in_start(sid, slot, width):
        for ct in range(8):
            pltpu.async_copy(
                tab_hbm.at[pl.ds(ct * 8, 8), pl.ds(sid * 128, width)],
                slabs[slot].at[pl.ds(ct * 8, 8), pl.ds(0, width)],
                gsems[slot])

    def slab_in_wait(slot, width):
        for ct in range(8):
            pltpu.make_async_copy(
                tab_hbm.at[pl.ds(0, 8), pl.ds(0, width)],
                slabs[slot].at[pl.ds(ct * 8, 8), pl.ds(0, width)],
                gsems[slot]).wait()

    def pair_out_start(sid, slot, width):
        pltpu.async_copy(pbufs[slot].at[pl.ds(0, width // 2)],
                         pairs_hbm.at[pl.ds(sid * 64, width // 2)],
                         csems[slot])

    def pair_out_wait(slot, width):
        pltpu.make_async_copy(pbufs[slot].at[pl.ds(0, width // 2)],
                              pairs_hbm.at[pl.ds(0, width // 2)],
                              csems[slot]).wait()

    def transpose_slab(slot, width):
        sl = slabs[slot]
        pb = pbufs[slot]
        rvecs = [iota16 + rg * 16 for rg in range(width // 16)]
        rhalfs = [jnp.right_shift(rv, 1) for rv in rvecs]

        @pl.loop(0, 16)
        def diag(d):
            pd = jnp.bitwise_and(iota16 + d, 15)
            for c0 in range(0, 64, 16):
                cperm = pd + c0
                hs_cp = halfsel + cperm
                for rg in range(width // 16):
                    v = plsc.load_gather(sl, [cperm, rvecs[rg]])
                    plsc.store_scatter(pb, [rhalfs[rg], hs_cp], v)

    for b in range(NB):
        slab_in_start(b * NW + wid, b, 128)

    @pl.loop(0, RT_ITER // NB)
    def outer(o):
        for b in range(NB):
            i = o * NB + b
            sid = i * NW + wid

            @pl.when(sid < RT_FULL)
            def _():
                slab_in_wait(b, 128)

                @pl.when(i >= NB)
                def _():
                    pair_out_wait(b, 128)
                transpose_slab(b, 128)
                pair_out_start(sid, b, 128)

                @pl.when(sid + NB * NW < RT_FULL)
                def _():
                    slab_in_start(sid + NB * NW, b, 128)

    for b in range(NB):
        pair_out_wait(b, 128)


def _b_body(idx_hbm, tab_hbm, tail_hbm, out_hbm, idxv, idxh, tailv,
            rows, tbufs, gsems, csems):
    wid = lax.axis_index("s") * 2 + lax.axis_index("c")
    iota16, P = _diag_vectors()
    bvecs = [iota16 + g * 16 for g in range(8)]

    pltpu.sync_copy(idx_hbm.at[wid], idxv)
    pltpu.sync_copy(tail_hbm, tailv)

    @pl.loop(0, HIST)
    def halve(h):
        for k in range(8):
            v = idxv[h, pl.ds(k * 16, 16)]
            idxh[h, pl.ds(k * 16, 16)] = jnp.right_shift(v, 1)

    def gather_start(h, slot):
        pltpu.async_copy(tab_hbm.at[idxh.at[h]], rows[slot], gsems[slot])

    def gather_wait(slot):
        pltpu.make_async_copy(tab_hbm.at[idxh.at[0]], rows[slot],
                              gsems[slot]).wait()

    def copyout_start(h, slot):
        for ct in range(8):
            pltpu.async_copy(tbufs[slot].at[pl.ds(ct * 8, 8)],
                             out_hbm.at[h, ct, wid], csems[slot])

    def copyout_wait(slot):
        for ct in range(8):
            pltpu.make_async_copy(tbufs[slot].at[pl.ds(ct * 8, 8)],
                                  out_hbm.at[0, ct, 0], csems[slot]).wait()

    def transpose_block(h, slot):
        rv = rows[slot]
        tb = tbufs[slot]
        lofs, tmask, trows = [], [], []
        for g in range(8):
            iv = idxv[h, pl.ds(g * 16, 16)]
            lofs.append(jnp.bitwise_and(iv, 1) * 64)
            tmask.append(iv >= TAIL_BASE)
            trows.append(jnp.maximum(iv - TAIL_BASE, 0))

        @pl.loop(0, 16)
        def diag(d):
            pd = jnp.bitwise_and(iota16 + d, 15)
            for c0 in range(0, 64, 16):
                cp = pd + c0
                for g in range(8):
                    v = plsc.load_gather(rv, [bvecs[g], lofs[g] + cp])
                    vt = plsc.load_gather(tailv, [trows[g], cp])
                    v = jnp.where(tmask[g], vt, v)
                    plsc.store_scatter(tb, [cp, bvecs[g]], v)

    for b in range(NB):
        gather_start(b, b)

    @pl.loop(0, HIST // NB)
    def group(g):
        base = g * NB
        for b in range(NB):
            h = base + b
            gather_wait(b)

            @pl.when(g > 0)
            def _():
                copyout_wait(b)
            transpose_block(h, b)
            copyout_start(h, b)

            @pl.when(g < HIST // NB - 1)
            def _():
                gather_start(h + NB, b)

    for b in range(NB):
        copyout_wait(b)


@functools.partial(
    pl.kernel,
    out_type=jax.ShapeDtypeStruct((VOCAB // 2, 128), jnp.float32),
    mesh=plsc.VectorSubcoreMesh(core_axis_name="c", subcore_axis_name="s"),
    compiler_params=pltpu.CompilerParams(needs_layout_passes=False,
                                         use_tc_tiling_on_sc=True),
    scratch_types=(
        [pltpu.VMEM((64, 128), jnp.float32) for _ in range(NB)]
        + [pltpu.VMEM((64, 128), jnp.float32) for _ in range(NB)]
        + [pltpu.SemaphoreType.DMA for _ in range(2 * NB)]
    ),
)
def _table_transpose_kernel(tab_hbm, pairs_hbm, *bufs):
    slabs = bufs[:NB]
    pbufs = bufs[NB:2 * NB]
    gsems = bufs[2 * NB:3 * NB]
    csems = bufs[3 * NB:]
    _a_body(tab_hbm, pairs_hbm, slabs, pbufs, gsems, csems)


@functools.partial(
    pl.kernel,
    out_type=jax.ShapeDtypeStruct((HIST, 8, BT, 8, 128), jnp.float32),
    mesh=plsc.VectorSubcoreMesh(core_axis_name="c", subcore_axis_name="s"),
    compiler_params=pltpu.CompilerParams(needs_layout_passes=False),
    scratch_types=(
        [pltpu.VMEM((HIST, 128), jnp.int32),
         pltpu.VMEM((HIST, 128), jnp.int32),
         pltpu.VMEM((64, 64), jnp.float32)]
        + [pltpu.VMEM((128, 128), jnp.float32) for _ in range(NB)]
        + [pltpu.VMEM((64, 128), jnp.float32) for _ in range(NB)]
        + [pltpu.SemaphoreType.DMA for _ in range(2 * NB)]
    ),
)
def _gather_kernel(idx_hbm, tab_hbm, tail_hbm, out_hbm, idxv, idxh, tailv,
                   *bufs):
    rows = bufs[:NB]
    tbufs = bufs[NB:2 * NB]
    gsems = bufs[2 * NB:3 * NB]
    csems = bufs[3 * NB:]
    _b_body(idx_hbm, tab_hbm, tail_hbm, out_hbm, idxv, idxh, tailv,
            rows, tbufs, gsems, csems)


@jax.jit
def kernel(indices, table):
    idx3 = (indices.astype(jnp.int32).T
            .reshape(HIST, BT, 128).transpose(1, 0, 2))
    pairs = _table_transpose_kernel(table.T)
    tail = lax.slice(table, (TAIL_BASE, 0), (VOCAB, EMBED_DIM))
    o5 = _gather_kernel(idx3, pairs, tail)
    return o5.transpose(2, 4, 0, 1, 3).reshape(BATCH, HIST, EMBED_DIM)


# speedup vs baseline: 3.7286x; 1.9918x over previous
import functools

import jax
import jax.numpy as jnp
from jax import lax
from jax.experimental import pallas as pl
from jax.experimental.pallas import tpu as pltpu
from jax.experimental.pallas import tpu_sc as plsc

VOCAB = 1000000
EMBED_DIM = 64
BATCH = 4096
HIST = 200

NW = 32
BT = BATCH // 128
NB = 2
RT_FULL = VOCAB // 128
RT_ITER = 246
TAIL_BASE = RT_FULL * 128


def _diag_vectors():
    iota16 = lax.iota(jnp.int32, 16)
    return iota16, [jnp.bitwise_and(iota16 + d, 15) for d in range(16)]


def _a_body(tab_hbm, pairs_hbm, slabs, pbufs, gsems, csems):
    wid = lax.axis_index("s") * 2 + lax.axis_index("c")
    iota16, P = _diag_vectors()
    halfsel = jnp.bitwise_and(iota16, 1) * 64

    def slab_in_start(sid, slot, width):
        for ct in range(8):
            pltpu.async_copy(
                tab_hbm.at[pl.ds(ct * 8, 8), pl.ds(sid * 128, width)],
                slabs[slot].at[pl.ds(ct * 8, 8), pl.ds(0, width)],
                gsems[slot])

    def slab_in_wait(slot, width):
        for ct in range(8):
            pltpu.make_async_copy(
                tab_hbm.at[pl.ds(0, 8), pl.ds(0, width)],
                slabs[slot].at[pl.ds(ct * 8, 8), pl.ds(0, width)],
                gsems[slot]).wait()

    def pair_out_start(sid, slot, width):
        pltpu.async_copy(pbufs[slot].at[pl.ds(0, width // 2)],
                         pairs_hbm.at[pl.ds(sid * 64, width // 2)],
                         csems[slot])

    def pair_out_wait(slot, width):
        pltpu.make_async_copy(pbufs[slot].at[pl.ds(0, width // 2)],
                              pairs_hbm.at[pl.ds(0, width // 2)],
                              csems[slot]).wait()

    def transpose_slab(slot, width):
        sl = slabs[slot]
        pb = pbufs[slot]
        rvecs = [iota16 + rg * 16 for rg in range(width // 16)]
        rhalfs = [jnp.right_shift(rv, 1) for rv in rvecs]

        @pl.loop(0, 16)
        def diag(d):
            pd = jnp.bitwise_and(iota16 + d, 15)
            for c0 in range(0, 64, 16):
                cperm = pd + c0
                hs_cp = halfsel + cperm
                vs = [plsc.load_gather(sl, [cperm, rvecs[rg]])
                      for rg in range(width // 16)]
                for rg in range(width // 16):
                    plsc.store_scatter(pb, [rhalfs[rg], hs_cp], vs[rg])

    for b in range(NB):
        slab_in_start(b * NW + wid, b, 128)

    @pl.loop(0, RT_ITER // NB)
    def outer(o):
        for b in range(NB):
            i = o * NB + b
            sid = i * NW + wid

            @pl.when(sid < RT_FULL)
            def _():
                slab_in_wait(b, 128)

                @pl.when(i >= NB)
                def _():
                    pair_out_wait(b, 128)
                transpose_slab(b, 128)
                pair_out_start(sid, b, 128)

                @pl.when(sid + NB * NW < RT_FULL)
                def _():
                    slab_in_start(sid + NB * NW, b, 128)

    for b in range(NB):
        pair_out_wait(b, 128)


def _b_body(idx_hbm, tab_hbm, tail_hbm, out_hbm, idxv, idxh, tailv,
            rows, tbufs, gsems, csems):
    wid = lax.axis_index("s") * 2 + lax.axis_index("c")
    iota16, P = _diag_vectors()
    bvecs = [iota16 + g * 16 for g in range(8)]

    pltpu.sync_copy(idx_hbm.at[wid], idxv)
    pltpu.sync_copy(tail_hbm, tailv)

    @pl.loop(0, HIST)
    def halve(h):
        for k in range(8):
            v = idxv[h, pl.ds(k * 16, 16)]
            idxh[h, pl.ds(k * 16, 16)] = jnp.right_shift(v, 1)

    def gather_start(h, slot):
        pltpu.async_copy(tab_hbm.at[idxh.at[h]], rows[slot], gsems[slot])

    def gather_wait(slot):
        pltpu.make_async_copy(tab_hbm.at[idxh.at[0]], rows[slot],
                              gsems[slot]).wait()

    def copyout_start(h, slot):
        for ct in range(8):
            pltpu.async_copy(tbufs[slot].at[pl.ds(ct * 8, 8)],
                             out_hbm.at[h, ct, wid], csems[slot])

    def copyout_wait(slot):
        for ct in range(8):
            pltpu.make_async_copy(tbufs[slot].at[pl.ds(ct * 8, 8)],
                                  out_hbm.at[0, ct, 0], csems[slot]).wait()

    def transpose_block(h, slot):
        rv = rows[slot]
        tb = tbufs[slot]
        lofs, tmask, trows = [], [], []
        anyt = None
        for g in range(8):
            iv = idxv[h, pl.ds(g * 16, 16)]
            lofs.append(jnp.bitwise_and(iv, 1) * 64)
            m = iv >= TAIL_BASE
            tmask.append(m)
            trows.append(jnp.maximum(iv - TAIL_BASE, 0))
            anyt = m if anyt is None else jnp.logical_or(anyt, m)
        has_tail = jnp.any(anyt)

        @pl.when(jnp.logical_not(has_tail))
        def _():
            @pl.loop(0, 16)
            def diag(d):
                pd = jnp.bitwise_and(iota16 + d, 15)
                for c0 in range(0, 64, 16):
                    cp = pd + c0
                    vs = [plsc.load_gather(rv, [bvecs[g], lofs[g] + cp])
                          for g in range(8)]
                    for g in range(8):
                        plsc.store_scatter(tb, [cp, bvecs[g]], vs[g])

        @pl.when(has_tail)
        def _():
            @pl.loop(0, 16)
            def diag(d):
                pd = jnp.bitwise_and(iota16 + d, 15)
                for c0 in range(0, 64, 16):
                    cp = pd + c0
                    vs = [plsc.load_gather(rv, [bvecs[g], lofs[g] + cp])
                          for g in range(8)]
                    vts = [plsc.load_gather(tailv, [trows[g], cp])
                           for g in range(8)]
                    for g in range(8):
                        v = jnp.where(tmask[g], vts[g], vs[g])
                        plsc.store_scatter(tb, [cp, bvecs[g]], v)

    for b in range(NB):
        gather_start(b, b)

    @pl.loop(0, HIST // NB)
    def group(g):
        base = g * NB
        for b in range(NB):
            h = base + b
            gather_wait(b)

            @pl.when(g > 0)
            def _():
                copyout_wait(b)
            transpose_block(h, b)
            copyout_start(h, b)

            @pl.when(g < HIST // NB - 1)
            def _():
                gather_start(h + NB, b)

    for b in range(NB):
        copyout_wait(b)


@functools.partial(
    pl.kernel,
    out_type=jax.ShapeDtypeStruct((VOCAB // 2, 128), jnp.float32),
    mesh=plsc.VectorSubcoreMesh(core_axis_name="c", subcore_axis_name="s"),
    compiler_params=pltpu.CompilerParams(needs_layout_passes=False,
                                         use_tc_tiling_on_sc=True),
    scratch_types=(
        [pltpu.VMEM((64, 128), jnp.float32) for _ in range(NB)]
        + [pltpu.VMEM((64, 128), jnp.float32) for _ in range(NB)]
        + [pltpu.SemaphoreType.DMA for _ in range(2 * NB)]
    ),
)
def _table_transpose_kernel(tab_hbm, pairs_hbm, *bufs):
    slabs = bufs[:NB]
    pbufs = bufs[NB:2 * NB]
    gsems = bufs[2 * NB:3 * NB]
    csems = bufs[3 * NB:]
    _a_body(tab_hbm, pairs_hbm, slabs, pbufs, gsems, csems)


@functools.partial(
    pl.kernel,
    out_type=jax.ShapeDtypeStruct((HIST, 8, BT, 8, 128), jnp.float32),
    mesh=plsc.VectorSubcoreMesh(core_axis_name="c", subcore_axis_name="s"),
    compiler_params=pltpu.CompilerParams(needs_layout_passes=False),
    scratch_types=(
        [pltpu.VMEM((HIST, 128), jnp.int32),
         pltpu.VMEM((HIST, 128), jnp.int32),
         pltpu.VMEM((64, 64), jnp.float32)]
        + [pltpu.VMEM((128, 128), jnp.float32) for _ in range(NB)]
        + [pltpu.VMEM((64, 128), jnp.float32) for _ in range(NB)]
        + [pltpu.SemaphoreType.DMA for _ in range(2 * NB)]
    ),
)
def _gather_kernel(idx_hbm, tab_hbm, tail_hbm, out_hbm, idxv, idxh, tailv,
                   *bufs):
    rows = bufs[:NB]
    tbufs = bufs[NB:2 * NB]
    gsems = bufs[2 * NB:3 * NB]
    csems = bufs[3 * NB:]
    _b_body(idx_hbm, tab_hbm, tail_hbm, out_hbm, idxv, idxh, tailv,
            rows, tbufs, gsems, csems)


@jax.jit
def kernel(indices, table):
    idx3 = (indices.astype(jnp.int32).T
            .reshape(HIST, BT, 128).transpose(1, 0, 2))
    pairs = _table_transpose_kernel(table.T)
    tail = lax.slice(table, (TAIL_BASE, 0), (VOCAB, EMBED_DIM))
    o5 = _gather_kernel(idx3, pairs, tail)
    return o5.transpose(2, 4, 0, 1, 3).reshape(BATCH, HIST, EMBED_DIM)


# speedup vs baseline: 4.1256x; 1.1065x over previous
import functools

import jax
import jax.numpy as jnp
from jax import lax
from jax.experimental import pallas as pl
from jax.experimental.pallas import tpu as pltpu
from jax.experimental.pallas import tpu_sc as plsc

VOCAB = 1000000
EMBED_DIM = 64
BATCH = 4096
HIST = 200

NW = 32
BT = BATCH // 128
NB = 2
NBA = 4
RT_FULL = VOCAB // 128
RT_ITER = 248
TAIL_BASE = RT_FULL * 128


def _diag_vectors():
    iota16 = lax.iota(jnp.int32, 16)
    return iota16, [jnp.bitwise_and(iota16 + d, 15) for d in range(16)]


def _a_body(tab_hbm, pairs_hbm, slabs, pbufs, gsems, csems):
    wid = lax.axis_index("s") * 2 + lax.axis_index("c")
    iota16, P = _diag_vectors()
    halfsel = jnp.bitwise_and(iota16, 1) * 64

    def slab_in_start(sid, slot, width):
        for ct in range(8):
            pltpu.async_copy(
                tab_hbm.at[pl.ds(ct * 8, 8), pl.ds(sid * 128, width)],
                slabs[slot].at[pl.ds(ct * 8, 8), pl.ds(0, width)],
                gsems[slot])

    def slab_in_wait(slot, width):
        for ct in range(8):
            pltpu.make_async_copy(
                tab_hbm.at[pl.ds(0, 8), pl.ds(0, width)],
                slabs[slot].at[pl.ds(ct * 8, 8), pl.ds(0, width)],
                gsems[slot]).wait()

    def pair_out_start(sid, slot, width):
        pltpu.async_copy(pbufs[slot].at[pl.ds(0, width // 2)],
                         pairs_hbm.at[pl.ds(sid * 64, width // 2)],
                         csems[slot])

    def pair_out_wait(slot, width):
        pltpu.make_async_copy(pbufs[slot].at[pl.ds(0, width // 2)],
                              pairs_hbm.at[pl.ds(0, width // 2)],
                              csems[slot]).wait()

    def transpose_slab(slot, width):
        sl = slabs[slot]
        pb = pbufs[slot]
        rvecs = [iota16 + rg * 16 for rg in range(width // 16)]
        rhalfs = [jnp.right_shift(rv, 1) for rv in rvecs]

        @pl.loop(0, 16)
        def diag(d):
            pd = jnp.bitwise_and(iota16 + d, 15)
            for c0 in range(0, 64, 16):
                cperm = pd + c0
                hs_cp = halfsel + cperm
                vs = [plsc.load_gather(sl, [cperm, rvecs[rg]])
                      for rg in range(width // 16)]
                for rg in range(width // 16):
                    plsc.store_scatter(pb, [rhalfs[rg], hs_cp], vs[rg])

    for b in range(NBA):
        slab_in_start(b * NW + wid, b, 128)

    @pl.loop(0, RT_ITER // NBA)
    def outer(o):
        for b in range(NBA):
            i = o * NBA + b
            sid = i * NW + wid

            @pl.when(sid < RT_FULL)
            def _():
                slab_in_wait(b, 128)

                @pl.when(i >= NBA)
                def _():
                    pair_out_wait(b, 128)
                transpose_slab(b, 128)
                pair_out_start(sid, b, 128)

                @pl.when(sid + NBA * NW < RT_FULL)
                def _():
                    slab_in_start(sid + NBA * NW, b, 128)

    for b in range(NBA):
        pair_out_wait(b, 128)


def _b_body(idx_hbm, tab_hbm, tail_hbm, out_hbm, idxv, idxh, tailv,
            rows, tbufs, gsems, csems):
    wid = lax.axis_index("s") * 2 + lax.axis_index("c")
    iota16, P = _diag_vectors()
    bvecs = [iota16 + g * 16 for g in range(8)]

    pltpu.sync_copy(idx_hbm.at[wid], idxv)
    pltpu.sync_copy(tail_hbm, tailv)

    @pl.loop(0, HIST)
    def halve(h):
        for k in range(8):
            v = idxv[h, pl.ds(k * 16, 16)]
            idxh[h, pl.ds(k * 16, 16)] = jnp.right_shift(v, 1)

    def gather_start(h, slot):
        pltpu.async_copy(tab_hbm.at[idxh.at[h]], rows[slot], gsems[slot])

    def gather_wait(slot):
        pltpu.make_async_copy(tab_hbm.at[idxh.at[0]], rows[slot],
                              gsems[slot]).wait()

    def copyout_start(h, slot):
        for ct in range(8):
            pltpu.async_copy(tbufs[slot].at[pl.ds(ct * 8, 8)],
                             out_hbm.at[h, ct, wid], csems[slot])

    def copyout_wait(slot):
        for ct in range(8):
            pltpu.make_async_copy(tbufs[slot].at[pl.ds(ct * 8, 8)],
                                  out_hbm.at[0, ct, 0], csems[slot]).wait()

    def transpose_block(h, slot):
        rv = rows[slot]
        tb = tbufs[slot]
        lofs, tmask, trows = [], [], []
        anyt = None
        for g in range(8):
            iv = idxv[h, pl.ds(g * 16, 16)]
            lofs.append(jnp.bitwise_and(iv, 1) * 64)
            m = iv >= TAIL_BASE
            tmask.append(m)
            trows.append(jnp.maximum(iv - TAIL_BASE, 0))
            anyt = m if anyt is None else jnp.logical_or(anyt, m)
        has_tail = jnp.any(anyt)

        @pl.when(jnp.logical_not(has_tail))
        def _():
            @pl.loop(0, 16)
            def diag(d):
                pd = jnp.bitwise_and(iota16 + d, 15)
                for c0 in range(0, 64, 16):
                    cp = pd + c0
                    vs = [plsc.load_gather(rv, [bvecs[g], lofs[g] + cp])
                          for g in range(8)]
                    for g in range(8):
                        plsc.store_scatter(tb, [cp, bvecs[g]], vs[g])

        @pl.when(has_tail)
        def _():
            @pl.loop(0, 16)
            def diag(d):
                pd = jnp.bitwise_and(iota16 + d, 15)
                for c0 in range(0, 64, 16):
                    cp = pd + c0
                    vs = [plsc.load_gather(rv, [bvecs[g], lofs[g] + cp])
                          for g in range(8)]
                    vts = [plsc.load_gather(tailv, [trows[g], cp])
                           for g in range(8)]
                    for g in range(8):
                        v = jnp.where(tmask[g], vts[g], vs[g])
                        plsc.store_scatter(tb, [cp, bvecs[g]], v)

    for b in range(NB):
        gather_start(b, b)

    @pl.loop(0, HIST // NB)
    def group(g):
        base = g * NB
        for b in range(NB):
            h = base + b
            gather_wait(b)

            @pl.when(g > 0)
            def _():
                copyout_wait(b)
            transpose_block(h, b)
            copyout_start(h, b)

            @pl.when(g < HIST // NB - 1)
            def _():
                gather_start(h + NB, b)

    for b in range(NB):
        copyout_wait(b)


@functools.partial(
    pl.kernel,
    out_type=jax.ShapeDtypeStruct((VOCAB // 2, 128), jnp.float32),
    mesh=plsc.VectorSubcoreMesh(core_axis_name="c", subcore_axis_name="s"),
    compiler_params=pltpu.CompilerParams(needs_layout_passes=False,
                                         use_tc_tiling_on_sc=True),
    scratch_types=(
        [pltpu.VMEM((64, 128), jnp.float32) for _ in range(NBA)]
        + [pltpu.VMEM((64, 128), jnp.float32) for _ in range(NBA)]
        + [pltpu.SemaphoreType.DMA for _ in range(2 * NBA)]
    ),
)
def _table_transpose_kernel(tab_hbm, pairs_hbm, *bufs):
    slabs = bufs[:NBA]
    pbufs = bufs[NBA:2 * NBA]
    gsems = bufs[2 * NBA:3 * NBA]
    csems = bufs[3 * NBA:]
    _a_body(tab_hbm, pairs_hbm, slabs, pbufs, gsems, csems)


@functools.partial(
    pl.kernel,
    out_type=jax.ShapeDtypeStruct((HIST, 8, BT, 8, 128), jnp.float32),
    mesh=plsc.VectorSubcoreMesh(core_axis_name="c", subcore_axis_name="s"),
    compiler_params=pltpu.CompilerParams(needs_layout_passes=False),
    scratch_types=(
        [pltpu.VMEM((HIST, 128), jnp.int32),
         pltpu.VMEM((HIST, 128), jnp.int32),
         pltpu.VMEM((64, 64), jnp.float32)]
        + [pltpu.VMEM((128, 128), jnp.float32) for _ in range(NB)]
        + [pltpu.VMEM((64, 128), jnp.float32) for _ in range(NB)]
        + [pltpu.SemaphoreType.DMA for _ in range(2 * NB)]
    ),
)
def _gather_kernel(idx_hbm, tab_hbm, tail_hbm, out_hbm, idxv, idxh, tailv,
                   *bufs):
    rows = bufs[:NB]
    tbufs = bufs[NB:2 * NB]
    gsems = bufs[2 * NB:3 * NB]
    csems = bufs[3 * NB:]
    _b_body(idx_hbm, tab_hbm, tail_hbm, out_hbm, idxv, idxh, tailv,
            rows, tbufs, gsems, csems)


@jax.jit
def kernel(indices, table):
    idx3 = (indices.astype(jnp.int32).T
            .reshape(HIST, BT, 128).transpose(1, 0, 2))
    pairs = _table_transpose_kernel(table.T)
    tail = lax.slice(table, (TAIL_BASE, 0), (VOCAB, EMBED_DIM))
    o5 = _gather_kernel(idx3, pairs, tail)
    return o5.transpose(2, 4, 0, 1, 3).reshape(BATCH, HIST, EMBED_DIM)


# speedup vs baseline: 4.4196x; 1.0713x over previous
import functools

import jax
import jax.numpy as jnp
from jax import lax
from jax.experimental import pallas as pl
from jax.experimental.pallas import tpu as pltpu
from jax.experimental.pallas import tpu_sc as plsc

VOCAB = 1000000
EMBED_DIM = 64
BATCH = 4096
HIST = 200

NW = 32
BT = BATCH // 128
NB = 4
NBA = 4
RT_FULL = VOCAB // 128
RT_ITER = 248
TAIL_BASE = RT_FULL * 128


def _diag_vectors():
    iota16 = lax.iota(jnp.int32, 16)
    return iota16, [jnp.bitwise_and(iota16 + d, 15) for d in range(16)]


def _a_body(tab_hbm, pairs_hbm, slabs, pbufs, gsems, csems):
    wid = lax.axis_index("s") * 2 + lax.axis_index("c")
    iota16, P = _diag_vectors()
    halfsel = jnp.bitwise_and(iota16, 1) * 64

    def slab_in_start(sid, slot, width):
        for ct in range(8):
            pltpu.async_copy(
                tab_hbm.at[pl.ds(ct * 8, 8), pl.ds(sid * 128, width)],
                slabs[slot].at[pl.ds(ct * 8, 8), pl.ds(0, width)],
                gsems[slot])

    def slab_in_wait(slot, width):
        for ct in range(8):
            pltpu.make_async_copy(
                tab_hbm.at[pl.ds(0, 8), pl.ds(0, width)],
                slabs[slot].at[pl.ds(ct * 8, 8), pl.ds(0, width)],
                gsems[slot]).wait()

    def pair_out_start(sid, slot, width):
        pltpu.async_copy(pbufs[slot].at[pl.ds(0, width // 2)],
                         pairs_hbm.at[pl.ds(sid * 64, width // 2)],
                         csems[slot])

    def pair_out_wait(slot, width):
        pltpu.make_async_copy(pbufs[slot].at[pl.ds(0, width // 2)],
                              pairs_hbm.at[pl.ds(0, width // 2)],
                              csems[slot]).wait()

    def transpose_slab(slot, width):
        sl = slabs[slot]
        pb = pbufs[slot]
        rvecs = [iota16 + rg * 16 for rg in range(width // 16)]
        rhalfs = [jnp.right_shift(rv, 1) for rv in rvecs]

        @pl.loop(0, 16)
        def diag(d):
            pd = jnp.bitwise_and(iota16 + d, 15)
            for c0 in range(0, 64, 16):
                cperm = pd + c0
                hs_cp = halfsel + cperm
                vs = [plsc.load_gather(sl, [cperm, rvecs[rg]])
                      for rg in range(width // 16)]
                for rg in range(width // 16):
                    plsc.store_scatter(pb, [rhalfs[rg], hs_cp], vs[rg])

    for b in range(NBA):
        slab_in_start(b * NW + wid, b, 128)

    @pl.loop(0, RT_ITER // NBA)
    def outer(o):
        for b in range(NBA):
            i = o * NBA + b
            sid = i * NW + wid

            @pl.when(sid < RT_FULL)
            def _():
                slab_in_wait(b, 128)

                @pl.when(i >= NBA)
                def _():
                    pair_out_wait(b, 128)
                transpose_slab(b, 128)
                pair_out_start(sid, b, 128)

                @pl.when(sid + NBA * NW < RT_FULL)
                def _():
                    slab_in_start(sid + NBA * NW, b, 128)

    for b in range(NBA):
        pair_out_wait(b, 128)


def _b_body(idx_hbm, tab_hbm, tail_hbm, out_hbm, tailv, ibufs, hbufs,
            rows, tbufs, isems, gsems, csems):
    wid = lax.axis_index("s") * 2 + lax.axis_index("c")
    iota16, P = _diag_vectors()
    bvecs = [iota16 + g * 16 for g in range(8)]

    pltpu.sync_copy(tail_hbm, tailv)

    def idx_start(h, slot):
        pltpu.async_copy(idx_hbm.at[wid, h], ibufs[slot], isems[slot])

    def idx_wait(slot):
        pltpu.make_async_copy(idx_hbm.at[0, 0], ibufs[slot],
                              isems[slot]).wait()

    def gather_start(slot):
        idx_wait(slot)
        for k in range(8):
            v = ibufs[slot][pl.ds(k * 16, 16)]
            hbufs[slot][pl.ds(k * 16, 16)] = jnp.right_shift(v, 1)
        pltpu.async_copy(tab_hbm.at[hbufs[slot]], rows[slot], gsems[slot])

    def gather_wait(slot):
        pltpu.make_async_copy(tab_hbm.at[hbufs[0]], rows[slot],
                              gsems[slot]).wait()

    def copyout_start(h, slot):
        for ct in range(8):
            pltpu.async_copy(tbufs[slot].at[pl.ds(ct * 8, 8)],
                             out_hbm.at[h, ct, wid], csems[slot])

    def copyout_wait(slot):
        for ct in range(8):
            pltpu.make_async_copy(tbufs[slot].at[pl.ds(ct * 8, 8)],
                                  out_hbm.at[0, ct, 0], csems[slot]).wait()

    def derive_select(slot):
        lofs, tmask, trows = [], [], []
        anyt = None
        for g in range(8):
            iv = ibufs[slot][pl.ds(g * 16, 16)]
            lofs.append(jnp.bitwise_and(iv, 1) * 64)
            m = iv >= TAIL_BASE
            tmask.append(m)
            trows.append(jnp.maximum(iv - TAIL_BASE, 0))
            anyt = m if anyt is None else jnp.logical_or(anyt, m)
        return lofs, tmask, trows, jnp.any(anyt)

    def transpose_block(sel, slot):
        rv = rows[slot]
        tb = tbufs[slot]
        lofs, tmask, trows, has_tail = sel

        @pl.when(jnp.logical_not(has_tail))
        def _():
            @pl.loop(0, 16)
            def diag(d):
                pd = jnp.bitwise_and(iota16 + d, 15)
                for c0 in range(0, 64, 16):
                    cp = pd + c0
                    vs = [plsc.load_gather(rv, [bvecs[g], lofs[g] + cp])
                          for g in range(8)]
                    for g in range(8):
                        plsc.store_scatter(tb, [cp, bvecs[g]], vs[g])

        @pl.when(has_tail)
        def _():
            @pl.loop(0, 16)
            def diag(d):
                pd = jnp.bitwise_and(iota16 + d, 15)
                for c0 in range(0, 64, 16):
                    cp = pd + c0
                    vs = [plsc.load_gather(rv, [bvecs[g], lofs[g] + cp])
                          for g in range(8)]
                    vts = [plsc.load_gather(tailv, [trows[g], cp])
                           for g in range(8)]
                    for g in range(8):
                        v = jnp.where(tmask[g], vts[g], vs[g])
                        plsc.store_scatter(tb, [cp, bvecs[g]], v)

    for b in range(NB):
        idx_start(b, b)
    for b in range(NB):
        gather_start(b)

    @pl.loop(0, HIST // NB)
    def group(g):
        base = g * NB
        for b in range(NB):
            h = base + b
            gather_wait(b)
            sel = derive_select(b)

            @pl.when(g < HIST // NB - 1)
            def _():
                idx_start(h + NB, b)

            @pl.when(g > 0)
            def _():
                copyout_wait(b)
            transpose_block(sel, b)
            copyout_start(h, b)

            @pl.when(g < HIST // NB - 1)
            def _():
                gather_start(b)

    for b in range(NB):
        copyout_wait(b)


@functools.partial(
    pl.kernel,
    out_type=jax.ShapeDtypeStruct((VOCAB // 2, 128), jnp.float32),
    mesh=plsc.VectorSubcoreMesh(core_axis_name="c", subcore_axis_name="s"),
    compiler_params=pltpu.CompilerParams(needs_layout_passes=False,
                                         use_tc_tiling_on_sc=True),
    scratch_types=(
        [pltpu.VMEM((64, 128), jnp.float32) for _ in range(NBA)]
        + [pltpu.VMEM((64, 128), jnp.float32) for _ in range(NBA)]
        + [pltpu.SemaphoreType.DMA for _ in range(2 * NBA)]
    ),
)
def _table_transpose_kernel(tab_hbm, pairs_hbm, *bufs):
    slabs = bufs[:NBA]
    pbufs = bufs[NBA:2 * NBA]
    gsems = bufs[2 * NBA:3 * NBA]
    csems = bufs[3 * NBA:]
    _a_body(tab_hbm, pairs_hbm, slabs, pbufs, gsems, csems)


@functools.partial(
    pl.kernel,
    out_type=jax.ShapeDtypeStruct((HIST, 8, BT, 8, 128), jnp.float32),
    mesh=plsc.VectorSubcoreMesh(core_axis_name="c", subcore_axis_name="s"),
    compiler_params=pltpu.CompilerParams(needs_layout_passes=False),
    scratch_types=(
        [pltpu.VMEM((64, 64), jnp.float32)]
        + [pltpu.VMEM((128,), jnp.int32) for _ in range(NB)]
        + [pltpu.VMEM((128,), jnp.int32) for _ in range(NB)]
        + [pltpu.VMEM((128, 128), jnp.float32) for _ in range(NB)]
        + [pltpu.VMEM((64, 128), jnp.float32) for _ in range(NB)]
        + [pltpu.SemaphoreType.DMA for _ in range(3 * NB)]
    ),
)
def _gather_kernel(idx_hbm, tab_hbm, tail_hbm, out_hbm, tailv, *bufs):
    ibufs = bufs[:NB]
    hbufs = bufs[NB:2 * NB]
    rows = bufs[2 * NB:3 * NB]
    tbufs = bufs[3 * NB:4 * NB]
    isems = bufs[4 * NB:5 * NB]
    gsems = bufs[5 * NB:6 * NB]
    csems = bufs[6 * NB:]
    _b_body(idx_hbm, tab_hbm, tail_hbm, out_hbm, tailv, ibufs, hbufs,
            rows, tbufs, isems, gsems, csems)


@jax.jit
def kernel(indices, table):
    idx3 = (indices.astype(jnp.int32).T
            .reshape(HIST, BT, 128).transpose(1, 0, 2))
    pairs = _table_transpose_kernel(table.T)
    tail = lax.slice(table, (TAIL_BASE, 0), (VOCAB, EMBED_DIM))
    o5 = _gather_kernel(idx3, pairs, tail)
    return o5.transpose(2, 4, 0, 1, 3).reshape(BATCH, HIST, EMBED_DIM)
